# Initial kernel scaffold; baseline (speedup 1.0000x reference)
import jax
import jax.numpy as jnp
from jax.experimental import pallas as pl


def kernel(distance):
    raise NotImplementedError("write your pallas kernel here")


# speedup vs baseline: 24.9465x; 24.9465x over previous
import jax
import jax.numpy as jnp
from jax.experimental import pallas as pl


def _body(x_ref, o_ref):
    x = x_ref[0]
    L, D = x.shape
    rng = jax.lax.broadcasted_iota(jnp.int32, (1, D), 1).astype(jnp.float32) - D / 2.0
    px = jnp.sum(x, axis=0, keepdims=True)
    px = px / jnp.sum(px)
    mean = jnp.sum(px * rng)
    std = jnp.sqrt(jnp.sum(px * (rng - mean) ** 2))
    new_idx = (rng - mean) / std + D / 2.0
    ii = new_idx.astype(jnp.int32)
    fl = jnp.clip(ii, 0, D - 1)
    ce = jnp.clip(ii + 1, 0, D - 1)
    w = new_idx - jnp.floor(new_idx)
    rows = jax.lax.broadcasted_iota(jnp.int32, (D, D), 0)
    g = jnp.where(rows == fl, 1.0 - w, 0.0) + jnp.where(rows == ce, w, 0.0)
    o_ref[0] = jax.lax.dot(x, g, precision=jax.lax.Precision.HIGHEST)


def kernel(distance):
    b, l, d = distance.shape
    return pl.pallas_call(
        _body,
        grid=(b,),
        in_specs=[pl.BlockSpec((1, l, d), lambda i: (i, 0, 0))],
        out_specs=pl.BlockSpec((1, l, d), lambda i: (i, 0, 0)),
        out_shape=jax.ShapeDtypeStruct((b, l, d), distance.dtype),
    )(distance)


# speedup vs baseline: 64.4855x; 2.5850x over previous
import jax
import jax.numpy as jnp
from jax.experimental import pallas as pl


def _body(x_ref, o_ref):
    x = x_ref[0]
    L, D = x.shape
    rng = jax.lax.broadcasted_iota(jnp.int32, (1, D), 1).astype(jnp.float32) - D / 2.0
    px = jnp.sum(x, axis=0, keepdims=True)
    px = px / jnp.sum(px)
    mean = jnp.sum(px * rng)
    std = jnp.sqrt(jnp.sum(px * (rng - mean) ** 2))
    new_idx = (rng - mean) / std + D / 2.0
    ii = new_idx.astype(jnp.int32)
    fl = jnp.clip(ii, 0, D - 1)
    ce = jnp.clip(ii + 1, 0, D - 1)
    w = new_idx - jnp.floor(new_idx)
    rows = jax.lax.broadcasted_iota(jnp.int32, (D, D), 0)
    g = jnp.where(rows == fl, 1.0 - w, 0.0) + jnp.where(rows == ce, w, 0.0)
    o_ref[0] = jax.lax.dot(
        x.astype(jnp.bfloat16),
        g.astype(jnp.bfloat16),
        preferred_element_type=jnp.float32,
    )


def kernel(distance):
    b, l, d = distance.shape
    return pl.pallas_call(
        _body,
        grid=(b,),
        in_specs=[pl.BlockSpec((1, l, d), lambda i: (i, 0, 0))],
        out_specs=pl.BlockSpec((1, l, d), lambda i: (i, 0, 0)),
        out_shape=jax.ShapeDtypeStruct((b, l, d), distance.dtype),
    )(distance)
